# Initial kernel scaffold; baseline (speedup 1.0000x reference)
import jax
import jax.numpy as jnp
from jax.experimental import pallas as pl


def kernel(responses, response_table, position_table):
    raise NotImplementedError("write your pallas kernel here")


# speedup vs baseline: 4.7772x; 4.7772x over previous
import jax
import jax.numpy as jnp
from jax import lax
from jax.experimental import pallas as pl
from jax.experimental.pallas import tpu as pltpu
from jax.experimental.pallas import tpu_sc as plsc

N_RESP = 100000
D = 32
S = 200
B = 4096
NC = 2
NS = 16
NW = NC * NS
TOTAL = B * S
PER_W = TOTAL // NW
CHUNK = 800
NSTREAM = 10
STREAM = CHUNK // NSTREAM
NCHUNK = PER_W // CHUNK
SEQ_PER_CHUNK = CHUNK // S
CHUNK_PER_SUPER = 8
NSUPER = NCHUNK // CHUNK_PER_SUPER
IDX_ROWS = CHUNK_PER_SUPER * NSTREAM


def _body(resp_hbm, table_hbm, pos_hbm, out_hbm, idx_v, rows_v, pos_v, sem):
    wid = lax.axis_index("s") * NC + lax.axis_index("c")
    pltpu.sync_copy(pos_hbm, pos_v)

    def super_body(sc, carry):
        irow = wid * (PER_W // STREAM) + sc * IDX_ROWS
        pltpu.sync_copy(resp_hbm.at[pl.ds(irow, IDX_ROWS)], idx_v)

        def chunk_body(k, carry2):
            base = wid * PER_W + sc * (CHUNK_PER_SUPER * CHUNK) + k * CHUNK
            copies = [
                pltpu.async_copy(
                    table_hbm.at[idx_v.at[k * NSTREAM + j]],
                    rows_v.at[pl.ds(j * STREAM, STREAM)],
                    sem,
                )
                for j in range(NSTREAM)
            ]
            for cp in copies:
                cp.wait()

            def add_body(s, inner):
                p0 = pos_v[s, pl.ds(0, 16)]
                p1 = pos_v[s, pl.ds(16, 16)]
                for q in range(SEQ_PER_CHUNK):
                    r = q * S + s
                    rows_v[r, pl.ds(0, 16)] += p0
                    rows_v[r, pl.ds(16, 16)] += p1
                return inner

            lax.fori_loop(0, S, add_body, 0)
            pltpu.sync_copy(rows_v, out_hbm.at[pl.ds(base, CHUNK)])
            return carry2

        lax.fori_loop(0, CHUNK_PER_SUPER, chunk_body, 0)
        return carry

    lax.fori_loop(0, NSUPER, super_body, 0)


_sc_kernel = pl.kernel(
    _body,
    out_type=jax.ShapeDtypeStruct((TOTAL, D), jnp.float32),
    mesh=plsc.VectorSubcoreMesh(
        core_axis_name="c", subcore_axis_name="s", num_cores=NC, num_subcores=NS
    ),
    scratch_types=[
        pltpu.VMEM((IDX_ROWS, STREAM), jnp.int32),
        pltpu.VMEM((CHUNK, D), jnp.float32),
        pltpu.VMEM((S, D), jnp.float32),
        pltpu.SemaphoreType.DMA,
    ],
    compiler_params=pltpu.CompilerParams(use_tc_tiling_on_sc=False),
)


def kernel(responses, response_table, position_table):
    resp2d = responses.astype(jnp.int32).reshape(TOTAL // STREAM, STREAM)
    out = _sc_kernel(resp2d, response_table, position_table)
    return out.reshape(B, S, D)


# speedup vs baseline: 5.2416x; 1.0972x over previous
import jax
import jax.numpy as jnp
from jax import lax
from jax.experimental import pallas as pl
from jax.experimental.pallas import tpu as pltpu
from jax.experimental.pallas import tpu_sc as plsc

N_RESP = 100000
D = 32
S = 200
B = 4096
NC = 2
NS = 16
NW = NC * NS
TOTAL = B * S
PER_W = TOTAL // NW
CHUNK = 800
NSTREAM = 10
STREAM = CHUNK // NSTREAM
NCHUNK = PER_W // CHUNK
NPAIR = NCHUNK // 2
SEQ_PER_CHUNK = CHUNK // S
IDX_ROWS = PER_W // STREAM


def _body(resp_hbm, table_hbm, pos_hbm, out_hbm,
          idx_v, rows0, rows1, pos_v, g0, g1, st0, st1):
    wid = lax.axis_index("s") * NC + lax.axis_index("c")
    base_w = wid * PER_W
    pltpu.sync_copy(pos_hbm, pos_v)
    pltpu.sync_copy(resp_hbm.at[pl.ds(wid * IDX_ROWS, IDX_ROWS)], idx_v)

    def fire_gather(c, buf, sem):
        for j in range(NSTREAM):
            pltpu.async_copy(
                table_hbm.at[idx_v.at[c * NSTREAM + j]],
                buf.at[pl.ds(j * STREAM, STREAM)],
                sem,
            )

    def drain_gather(buf, sem):
        pltpu.make_async_copy(out_hbm.at[pl.ds(0, CHUNK)], buf, sem).wait()

    def drain_store(buf, sem):
        pltpu.make_async_copy(buf, out_hbm.at[pl.ds(0, CHUNK)], sem).wait()

    def add_pos(buf):
        def add_body(s, inner):
            p0 = pos_v[s, pl.ds(0, 16)]
            p1 = pos_v[s, pl.ds(16, 16)]
            for q in range(SEQ_PER_CHUNK):
                r = q * S + s
                buf[r, pl.ds(0, 16)] += p0
                buf[r, pl.ds(16, 16)] += p1
            return inner

        lax.fori_loop(0, S, add_body, 0)

    fire_gather(0, rows0, g0)

    def pair_body(i, carry):
        c0 = 2 * i
        c1 = 2 * i + 1

        @pl.when(i > 0)
        def _():
            drain_store(rows1, st1)

        fire_gather(c1, rows1, g1)
        drain_gather(rows0, g0)
        add_pos(rows0)
        pltpu.async_copy(rows0, out_hbm.at[pl.ds(base_w + c0 * CHUNK, CHUNK)], st0)
        drain_gather(rows1, g1)
        drain_store(rows0, st0)

        @pl.when(i < NPAIR - 1)
        def _():
            fire_gather(c0 + 2, rows0, g0)

        add_pos(rows1)
        pltpu.async_copy(rows1, out_hbm.at[pl.ds(base_w + c1 * CHUNK, CHUNK)], st1)
        return carry

    lax.fori_loop(0, NPAIR, pair_body, 0)
    drain_store(rows1, st1)


_sc_kernel = pl.kernel(
    _body,
    out_type=jax.ShapeDtypeStruct((TOTAL, D), jnp.float32),
    mesh=plsc.VectorSubcoreMesh(
        core_axis_name="c", subcore_axis_name="s", num_cores=NC, num_subcores=NS
    ),
    scratch_types=[
        pltpu.VMEM((IDX_ROWS, STREAM), jnp.int32),
        pltpu.VMEM((CHUNK, D), jnp.float32),
        pltpu.VMEM((CHUNK, D), jnp.float32),
        pltpu.VMEM((S, D), jnp.float32),
        pltpu.SemaphoreType.DMA,
        pltpu.SemaphoreType.DMA,
        pltpu.SemaphoreType.DMA,
        pltpu.SemaphoreType.DMA,
    ],
    compiler_params=pltpu.CompilerParams(use_tc_tiling_on_sc=False),
)


def kernel(responses, response_table, position_table):
    resp2d = responses.astype(jnp.int32).reshape(TOTAL // STREAM, STREAM)
    out = _sc_kernel(resp2d, response_table, position_table)
    return out.reshape(B, S, D)
